# baseline (speedup 1.0000x reference)
import jax
import jax.numpy as jnp
from jax import lax
from jax.experimental import pallas as pl
from jax.experimental.pallas import tpu as pltpu
from jax.experimental.pallas import tpu_sc as plsc

B = 4096
N = 50
NP = 56
D = 64
NC = 2
NS = 16
NW = NC * NS

BW = B // NW
RPB = 4
ROUNDS = BW // RPB
HOP_ROWS = B * NP


def _sc_gather_body(item_emb, user_emb, s1p, s2p, s3p, tgt,
                    g1, g2, g3, gt, idx_v, tidx_v, buf, sem, wsem):
    wid = lax.axis_index("s") * NC + lax.axis_index("c")
    b0 = wid * BW

    def do_hop(table, idx_hbm, out_hbm):
        pltpu.sync_copy(idx_hbm.at[pl.ds(b0, BW)], idx_v)

        def round_step(r, carry):
            cps = []
            for k in range(RPB):
                cps.append(pltpu.async_copy(
                    table.at[idx_v.at[r * RPB + k]],
                    buf.at[pl.ds(k * NP, NP)], sem))
            for cp in cps:
                cp.wait()
            pltpu.async_copy(
                buf, out_hbm.at[pl.ds((b0 + r * RPB) * NP, RPB * NP)],
                wsem).wait()
            return carry

        lax.fori_loop(0, ROUNDS, round_step, 0)

    do_hop(item_emb, s1p, g1)
    do_hop(user_emb, s2p, g2)
    do_hop(item_emb, s3p, g3)

    pltpu.sync_copy(tgt.at[pl.ds(b0, BW)], tidx_v)
    pltpu.async_copy(user_emb.at[tidx_v], buf.at[pl.ds(0, BW)], sem).wait()
    pltpu.async_copy(buf.at[pl.ds(0, BW)], gt.at[pl.ds(b0, BW)], wsem).wait()


def _make_sc_gather():
    mesh = plsc.VectorSubcoreMesh(core_axis_name="c", subcore_axis_name="s")
    return pl.kernel(
        _sc_gather_body,
        out_type=(
            jax.ShapeDtypeStruct((HOP_ROWS, D), jnp.float32),
            jax.ShapeDtypeStruct((HOP_ROWS, D), jnp.float32),
            jax.ShapeDtypeStruct((HOP_ROWS, D), jnp.float32),
            jax.ShapeDtypeStruct((B, D), jnp.float32),
        ),
        mesh=mesh,
        scratch_types=[
            pltpu.VMEM((BW, NP), jnp.int32),
            pltpu.VMEM((BW,), jnp.int32),
            pltpu.VMEM((RPB * NP, D), jnp.float32),
            pltpu.SemaphoreType.DMA,
            pltpu.SemaphoreType.DMA,
        ],
        compiler_params=pltpu.CompilerParams(use_tc_tiling_on_sc=False),
    )


BB = 128


def _tc_gat_body(g1_ref, g2_ref, g3_ref, gt_ref, aw_ref, av_ref, rw_ref, out_ref):
    aw = aw_ref[...]
    av = av_ref[...]
    rw = rw_ref[...]

    def gat(embf):
        t = jnp.tanh(jnp.dot(embf, aw, preferred_element_type=jnp.float32))
        s = jnp.dot(t, av, preferred_element_type=jnp.float32)
        s3 = s.reshape(BB, NP, 1)
        nidx = lax.broadcasted_iota(jnp.int32, (BB, NP, 1), 1)
        s3 = jnp.where(nidx < N, s3, -1e30)
        m = jnp.max(s3, axis=1, keepdims=True)
        e = jnp.exp(s3 - m)
        alpha = e / jnp.sum(e, axis=1, keepdims=True)
        w = embf * alpha.reshape(BB * NP, 1)
        return jnp.sum(w.reshape(BB, NP, D), axis=1)

    agg1 = gat(g1_ref[...])
    agg2 = gat(g2_ref[...])
    agg3 = gat(g3_ref[...])
    tgt = gt_ref[...]

    acc = (jnp.dot(agg1, rw[0:D], preferred_element_type=jnp.float32)
           + jnp.dot(agg2, rw[D:2 * D], preferred_element_type=jnp.float32)
           + jnp.dot(agg3, rw[2 * D:3 * D], preferred_element_type=jnp.float32)
           + jnp.dot(tgt, rw[3 * D:4 * D], preferred_element_type=jnp.float32))
    out_ref[...] = jnp.tanh(acc)


def _tc_gat(g1, g2, g3, gt, att_w, av_col, refine_w):
    grid = (B // BB,)
    hop_spec = pl.BlockSpec((BB * NP, D), lambda i: (i, 0))
    return pl.pallas_call(
        _tc_gat_body,
        grid=grid,
        in_specs=[
            hop_spec, hop_spec, hop_spec,
            pl.BlockSpec((BB, D), lambda i: (i, 0)),
            pl.BlockSpec((D, D), lambda i: (0, 0)),
            pl.BlockSpec((D, 1), lambda i: (0, 0)),
            pl.BlockSpec((4 * D, D), lambda i: (0, 0)),
        ],
        out_specs=pl.BlockSpec((BB, D), lambda i: (i, 0)),
        out_shape=jax.ShapeDtypeStruct((B, D), jnp.float32),
    )(g1, g2, g3, gt, att_w, av_col, refine_w)


def kernel(target_ids, support_1st, support_2nd, support_3rd,
           user_emb, item_emb, att_w, att_v, refine_w):
    pad = ((0, 0), (0, NP - N))
    s1p = jnp.pad(support_1st, pad)
    s2p = jnp.pad(support_2nd, pad)
    s3p = jnp.pad(support_3rd, pad)

    g1, g2, g3, gt = _make_sc_gather()(item_emb, user_emb, s1p, s2p, s3p,
                                       target_ids)
    return _tc_gat(g1, g2, g3, gt, att_w, att_v.reshape(D, 1), refine_w)


# speedup vs baseline: 1.0183x; 1.0183x over previous
import jax
import jax.numpy as jnp
from jax import lax
from jax.experimental import pallas as pl
from jax.experimental.pallas import tpu as pltpu
from jax.experimental.pallas import tpu_sc as plsc

B = 4096
N = 50
NP = 56
D = 64
NC = 2
NS = 16
NW = NC * NS

BW = B // NW
CHUNK = 128
HOP_ROWS = B * NP
PER_W = HOP_ROWS // NW
N_CHUNKS = PER_W // CHUNK


def _gather_hop(table, idx_hbm, out_hbm, wid, idx_v, rows_v, sem, wsem):
    pltpu.sync_copy(idx_hbm.at[wid], idx_v)

    def step(c, carry):
        pltpu.async_copy(table.at[idx_v.at[c]], rows_v, sem).wait()
        pltpu.async_copy(
            rows_v, out_hbm.at[pl.ds(wid * PER_W + c * CHUNK, CHUNK)],
            wsem).wait()
        return carry

    lax.fori_loop(0, N_CHUNKS, step, 0)


def _sc_user_body(user_emb, uidx, tgt, g2, gt, idx_v, tidx_v, rows_v, sem, wsem):
    wid = lax.axis_index("s") * NC + lax.axis_index("c")
    _gather_hop(user_emb, uidx, g2, wid, idx_v, rows_v, sem, wsem)
    pltpu.sync_copy(tgt.at[pl.ds(wid * BW, BW)], tidx_v)
    pltpu.async_copy(user_emb.at[tidx_v], rows_v.at[pl.ds(0, BW)], sem).wait()
    pltpu.async_copy(rows_v.at[pl.ds(0, BW)], gt.at[pl.ds(wid * BW, BW)], wsem).wait()


def _sc_item_body(item_emb, i1idx, i3idx, g1, g3, idx_v, rows_v, sem, wsem):
    wid = lax.axis_index("s") * NC + lax.axis_index("c")
    _gather_hop(item_emb, i1idx, g1, wid, idx_v, rows_v, sem, wsem)
    _gather_hop(item_emb, i3idx, g3, wid, idx_v, rows_v, sem, wsem)


def _make_sc_user():
    mesh = plsc.VectorSubcoreMesh(core_axis_name="c", subcore_axis_name="s")
    return pl.kernel(
        _sc_user_body,
        out_type=(
            jax.ShapeDtypeStruct((HOP_ROWS, D), jnp.float32),
            jax.ShapeDtypeStruct((B, D), jnp.float32),
        ),
        mesh=mesh,
        scratch_types=[
            pltpu.VMEM((N_CHUNKS, CHUNK), jnp.int32),
            pltpu.VMEM((BW,), jnp.int32),
            pltpu.VMEM((CHUNK, D), jnp.float32),
            pltpu.SemaphoreType.DMA,
            pltpu.SemaphoreType.DMA,
        ],
        compiler_params=pltpu.CompilerParams(use_tc_tiling_on_sc=False),
    )


def _make_sc_item():
    mesh = plsc.VectorSubcoreMesh(core_axis_name="c", subcore_axis_name="s")
    return pl.kernel(
        _sc_item_body,
        out_type=(
            jax.ShapeDtypeStruct((HOP_ROWS, D), jnp.float32),
            jax.ShapeDtypeStruct((HOP_ROWS, D), jnp.float32),
        ),
        mesh=mesh,
        scratch_types=[
            pltpu.VMEM((N_CHUNKS, CHUNK), jnp.int32),
            pltpu.VMEM((CHUNK, D), jnp.float32),
            pltpu.SemaphoreType.DMA,
            pltpu.SemaphoreType.DMA,
        ],
        compiler_params=pltpu.CompilerParams(use_tc_tiling_on_sc=False),
    )


BB = 128


def _gat_block(embf, aw, av):
    t = jnp.tanh(jnp.dot(embf, aw, preferred_element_type=jnp.float32))
    s = jnp.dot(t, av, preferred_element_type=jnp.float32)
    s3 = s.reshape(BB, NP, 1)
    nidx = lax.broadcasted_iota(jnp.int32, (BB, NP, 1), 1)
    s3 = jnp.where(nidx < N, s3, -1e30)
    m = jnp.max(s3, axis=1, keepdims=True)
    e = jnp.exp(s3 - m)
    alpha = e / jnp.sum(e, axis=1, keepdims=True)
    w = embf * alpha.reshape(BB * NP, 1)
    return jnp.sum(w.reshape(BB, NP, D), axis=1)


def _tc_user_body(g2_ref, gt_ref, aw_ref, av_ref, rw_ref, part_ref):
    aw = aw_ref[...]
    av = av_ref[...]
    rw = rw_ref[...]
    agg2 = _gat_block(g2_ref[...], aw, av)
    tgt = gt_ref[...]
    part_ref[...] = (
        jnp.dot(agg2, rw[D:2 * D], preferred_element_type=jnp.float32)
        + jnp.dot(tgt, rw[3 * D:4 * D], preferred_element_type=jnp.float32))


def _tc_item_body(g1_ref, g3_ref, part_ref, aw_ref, av_ref, rw_ref, out_ref):
    aw = aw_ref[...]
    av = av_ref[...]
    rw = rw_ref[...]
    agg1 = _gat_block(g1_ref[...], aw, av)
    agg3 = _gat_block(g3_ref[...], aw, av)
    acc = (jnp.dot(agg1, rw[0:D], preferred_element_type=jnp.float32)
           + jnp.dot(agg3, rw[2 * D:3 * D], preferred_element_type=jnp.float32)
           + part_ref[...])
    out_ref[...] = jnp.tanh(acc)


_HOP_SPEC = pl.BlockSpec((BB * NP, D), lambda i: (i, 0))
_ROW_SPEC = pl.BlockSpec((BB, D), lambda i: (i, 0))
_W_SPECS = [
    pl.BlockSpec((D, D), lambda i: (0, 0)),
    pl.BlockSpec((D, 1), lambda i: (0, 0)),
    pl.BlockSpec((4 * D, D), lambda i: (0, 0)),
]


def _tc_user(g2, gt, att_w, av_col, refine_w):
    return pl.pallas_call(
        _tc_user_body,
        grid=(B // BB,),
        in_specs=[_HOP_SPEC, _ROW_SPEC] + _W_SPECS,
        out_specs=_ROW_SPEC,
        out_shape=jax.ShapeDtypeStruct((B, D), jnp.float32),
    )(g2, gt, att_w, av_col, refine_w)


def _tc_item(g1, g3, part, att_w, av_col, refine_w):
    return pl.pallas_call(
        _tc_item_body,
        grid=(B // BB,),
        in_specs=[_HOP_SPEC, _HOP_SPEC, _ROW_SPEC] + _W_SPECS,
        out_specs=_ROW_SPEC,
        out_shape=jax.ShapeDtypeStruct((B, D), jnp.float32),
    )(g1, g3, part, att_w, av_col, refine_w)


def kernel(target_ids, support_1st, support_2nd, support_3rd,
           user_emb, item_emb, att_w, att_v, refine_w):
    pad = ((0, 0), (0, NP - N))
    i1idx = jnp.pad(support_1st, pad).reshape(NW, N_CHUNKS, CHUNK)
    uidx = jnp.pad(support_2nd, pad).reshape(NW, N_CHUNKS, CHUNK)
    i3idx = jnp.pad(support_3rd, pad).reshape(NW, N_CHUNKS, CHUNK)

    g2, gt = _make_sc_user()(user_emb, uidx, target_ids)
    g1, g3 = _make_sc_item()(item_emb, i1idx, i3idx)

    av_col = att_v.reshape(D, 1)
    part = _tc_user(g2, gt, att_w, av_col, refine_w)
    return _tc_item(g1, g3, part, att_w, av_col, refine_w)


# speedup vs baseline: 1.9473x; 1.9123x over previous
import jax
import jax.numpy as jnp
from jax import lax
from jax.experimental import pallas as pl
from jax.experimental.pallas import tpu as pltpu
from jax.experimental.pallas import tpu_sc as plsc

B = 4096
N = 50
NP = 56
D = 64
NC = 2
NS = 16
NW = NC * NS

BW = B // NW
CHUNK = 128
HOP_ROWS = B * NP
PER_W = HOP_ROWS // NW
N_CHUNKS = PER_W // CHUNK


def _gather_hop(table, idx_hbm, out_hbm, wid, idx_v, rows_v, sem, wsem):
    pltpu.sync_copy(idx_hbm.at[wid], idx_v)

    def step(c, carry):
        pltpu.async_copy(table.at[idx_v.at[c]], rows_v, sem).wait()
        pltpu.async_copy(
            rows_v, out_hbm.at[pl.ds(wid * PER_W + c * CHUNK, CHUNK)],
            wsem).wait()
        return carry

    lax.fori_loop(0, N_CHUNKS, step, 0)


def _sc_user_body(user_emb, uidx, tgt, g2, gt, idx_v, tidx_v, rows_v, sem, wsem):
    wid = lax.axis_index("s") * NC + lax.axis_index("c")
    _gather_hop(user_emb, uidx, g2, wid, idx_v, rows_v, sem, wsem)
    pltpu.sync_copy(tgt.at[pl.ds(wid * BW, BW)], tidx_v)
    pltpu.async_copy(user_emb.at[tidx_v], rows_v.at[pl.ds(0, BW)], sem).wait()
    pltpu.async_copy(rows_v.at[pl.ds(0, BW)], gt.at[pl.ds(wid * BW, BW)], wsem).wait()


def _sc_item_body(item_emb, i1idx, i3idx, g1, g3, idx_v, rows_v, sem, wsem):
    wid = lax.axis_index("s") * NC + lax.axis_index("c")
    _gather_hop(item_emb, i1idx, g1, wid, idx_v, rows_v, sem, wsem)
    _gather_hop(item_emb, i3idx, g3, wid, idx_v, rows_v, sem, wsem)


def _make_sc_user():
    mesh = plsc.VectorSubcoreMesh(core_axis_name="c", subcore_axis_name="s")
    return pl.kernel(
        _sc_user_body,
        out_type=(
            jax.ShapeDtypeStruct((HOP_ROWS, D), jnp.float32),
            jax.ShapeDtypeStruct((B, D), jnp.float32),
        ),
        mesh=mesh,
        scratch_types=[
            pltpu.VMEM((N_CHUNKS, CHUNK), jnp.int32),
            pltpu.VMEM((BW,), jnp.int32),
            pltpu.VMEM((CHUNK, D), jnp.float32),
            pltpu.SemaphoreType.DMA,
            pltpu.SemaphoreType.DMA,
        ],
        compiler_params=pltpu.CompilerParams(use_tc_tiling_on_sc=False),
    )


def _make_sc_item():
    mesh = plsc.VectorSubcoreMesh(core_axis_name="c", subcore_axis_name="s")
    return pl.kernel(
        _sc_item_body,
        out_type=(
            jax.ShapeDtypeStruct((HOP_ROWS, D), jnp.float32),
            jax.ShapeDtypeStruct((HOP_ROWS, D), jnp.float32),
        ),
        mesh=mesh,
        scratch_types=[
            pltpu.VMEM((N_CHUNKS, CHUNK), jnp.int32),
            pltpu.VMEM((CHUNK, D), jnp.float32),
            pltpu.SemaphoreType.DMA,
            pltpu.SemaphoreType.DMA,
        ],
        compiler_params=pltpu.CompilerParams(use_tc_tiling_on_sc=False),
    )


BB = 128


def _gat_block(embf, aw, av):
    t = jnp.tanh(jnp.dot(embf, aw, preferred_element_type=jnp.float32))
    s = jnp.dot(t, av, preferred_element_type=jnp.float32)
    s3 = s.reshape(BB, NP, 1)
    nidx = lax.broadcasted_iota(jnp.int32, (BB, NP, 1), 1)
    s3 = jnp.where(nidx < N, s3, -1e30)
    m = jnp.max(s3, axis=1, keepdims=True)
    e = jnp.exp(s3 - m)
    alpha = e / jnp.sum(e, axis=1, keepdims=True)
    w = embf * alpha.reshape(BB * NP, 1)
    return jnp.sum(w.reshape(BB, NP, D), axis=1)


def _tc_user_body(g2_ref, gt_ref, aw_ref, av_ref, rw_ref, part_ref):
    aw = aw_ref[...]
    av = av_ref[...]
    rw = rw_ref[...]
    agg2 = _gat_block(g2_ref[...], aw, av)
    tgt = gt_ref[...]
    part_ref[...] = (
        jnp.dot(agg2, rw[D:2 * D], preferred_element_type=jnp.float32)
        + jnp.dot(tgt, rw[3 * D:4 * D], preferred_element_type=jnp.float32))


def _tc_item_body(g1_ref, g3_ref, part_ref, aw_ref, av_ref, rw_ref, out_ref):
    aw = aw_ref[...]
    av = av_ref[...]
    rw = rw_ref[...]
    agg1 = _gat_block(g1_ref[...], aw, av)
    agg3 = _gat_block(g3_ref[...], aw, av)
    acc = (jnp.dot(agg1, rw[0:D], preferred_element_type=jnp.float32)
           + jnp.dot(agg3, rw[2 * D:3 * D], preferred_element_type=jnp.float32)
           + part_ref[...])
    out_ref[...] = jnp.tanh(acc)


_HOP_SPEC = pl.BlockSpec((BB * NP, D), lambda i: (i, 0))
_ROW_SPEC = pl.BlockSpec((BB, D), lambda i: (i, 0))
_W_SPECS = [
    pl.BlockSpec((D, D), lambda i: (0, 0)),
    pl.BlockSpec((D, 1), lambda i: (0, 0)),
    pl.BlockSpec((4 * D, D), lambda i: (0, 0)),
]


def _tc_user(g2, gt, att_w, av_col, refine_w):
    return pl.pallas_call(
        _tc_user_body,
        grid=(B // BB,),
        in_specs=[_HOP_SPEC, _ROW_SPEC] + _W_SPECS,
        out_specs=_ROW_SPEC,
        out_shape=jax.ShapeDtypeStruct((B, D), jnp.float32),
    )(g2, gt, att_w, av_col, refine_w)


def _tc_item(g1, g3, part, att_w, av_col, refine_w):
    return pl.pallas_call(
        _tc_item_body,
        grid=(B // BB,),
        in_specs=[_HOP_SPEC, _HOP_SPEC, _ROW_SPEC] + _W_SPECS,
        out_specs=_ROW_SPEC,
        out_shape=jax.ShapeDtypeStruct((B, D), jnp.float32),
    )(g1, g3, part, att_w, av_col, refine_w)


def kernel(target_ids, support_1st, support_2nd, support_3rd,
           user_emb, item_emb, att_w, att_v, refine_w):
    def pad_idx(s):
        return jnp.concatenate([s, s[:, :NP - N]], axis=1).reshape(
            NW, N_CHUNKS, CHUNK)

    i1idx = pad_idx(support_1st)
    uidx = pad_idx(support_2nd)
    i3idx = pad_idx(support_3rd)

    g2, gt = _make_sc_user()(user_emb, uidx, target_ids)
    g1, g3 = _make_sc_item()(item_emb, i1idx, i3idx)

    av_col = att_v.reshape(D, 1)
    part = _tc_user(g2, gt, att_w, av_col, refine_w)
    return _tc_item(g1, g3, part, att_w, av_col, refine_w)


# speedup vs baseline: 2.2510x; 1.1560x over previous
import jax
import jax.numpy as jnp
from jax import lax
from jax.experimental import pallas as pl
from jax.experimental.pallas import tpu as pltpu
from jax.experimental.pallas import tpu_sc as plsc

B = 4096
N = 50
NP = 56
D = 64
D2 = 2 * D
NC = 2
NS = 16
NW = NC * NS

BW = B // NW
CHUNK = 128
HOP_ROWS = B * NP
PER_W = HOP_ROWS // NW
N_CHUNKS = PER_W // CHUNK


def _sc_gather_body(combo, i1idx, uidx, i3idx, tgt,
                    g1, g2, g3, gt, idx_v, tidx_v, rows_v, sem, wsem):
    wid = lax.axis_index("s") * NC + lax.axis_index("c")

    def do_hop(idx_hbm, out_hbm):
        pltpu.sync_copy(idx_hbm.at[wid], idx_v)

        def step(c, carry):
            pltpu.async_copy(combo.at[idx_v.at[c]], rows_v, sem).wait()
            pltpu.async_copy(
                rows_v, out_hbm.at[pl.ds(wid * PER_W + c * CHUNK, CHUNK)],
                wsem).wait()
            return carry

        lax.fori_loop(0, N_CHUNKS, step, 0)

    do_hop(i1idx, g1)
    do_hop(uidx, g2)
    do_hop(i3idx, g3)

    pltpu.sync_copy(tgt.at[pl.ds(wid * BW, BW)], tidx_v)
    pltpu.async_copy(combo.at[tidx_v], rows_v.at[pl.ds(0, BW)], sem).wait()
    pltpu.async_copy(rows_v.at[pl.ds(0, BW)], gt.at[pl.ds(wid * BW, BW)],
                     wsem).wait()


def _make_sc_gather():
    mesh = plsc.VectorSubcoreMesh(core_axis_name="c", subcore_axis_name="s")
    return pl.kernel(
        _sc_gather_body,
        out_type=(
            jax.ShapeDtypeStruct((HOP_ROWS, D2), jnp.float32),
            jax.ShapeDtypeStruct((HOP_ROWS, D2), jnp.float32),
            jax.ShapeDtypeStruct((HOP_ROWS, D2), jnp.float32),
            jax.ShapeDtypeStruct((B, D2), jnp.float32),
        ),
        mesh=mesh,
        scratch_types=[
            pltpu.VMEM((N_CHUNKS, CHUNK), jnp.int32),
            pltpu.VMEM((BW,), jnp.int32),
            pltpu.VMEM((CHUNK, D2), jnp.float32),
            pltpu.SemaphoreType.DMA,
            pltpu.SemaphoreType.DMA,
        ],
        compiler_params=pltpu.CompilerParams(use_tc_tiling_on_sc=True),
    )


BB = 128


def _gat_block(embf, aw_h, av):
    t = jnp.tanh(jnp.dot(embf, aw_h, preferred_element_type=jnp.float32))
    s = jnp.dot(t, av, preferred_element_type=jnp.float32)
    s3 = s.reshape(BB, NP, 1)
    nidx = lax.broadcasted_iota(jnp.int32, (BB, NP, 1), 1)
    s3 = jnp.where(nidx < N, s3, -1e30)
    m = jnp.max(s3, axis=1, keepdims=True)
    e = jnp.exp(s3 - m)
    alpha = e / jnp.sum(e, axis=1, keepdims=True)
    w = embf * alpha.reshape(BB * NP, 1)
    return jnp.sum(w.reshape(BB, NP, D2), axis=1)


def _tc_gat_body(g1_ref, g2_ref, g3_ref, gt_ref,
                 awi_ref, awu_ref, av_ref, rwp_ref, out_ref):
    awi = awi_ref[...]
    awu = awu_ref[...]
    av = av_ref[...]
    rwp = rwp_ref[...]

    agg1 = _gat_block(g1_ref[...], awi, av)
    agg2 = _gat_block(g2_ref[...], awu, av)
    agg3 = _gat_block(g3_ref[...], awi, av)
    tgt = gt_ref[...]

    acc = (jnp.dot(agg1, rwp[0], preferred_element_type=jnp.float32)
           + jnp.dot(agg2, rwp[1], preferred_element_type=jnp.float32)
           + jnp.dot(agg3, rwp[2], preferred_element_type=jnp.float32)
           + jnp.dot(tgt, rwp[3], preferred_element_type=jnp.float32))
    out_ref[...] = jnp.tanh(acc)


def _tc_gat(g1, g2, g3, gt, awi, awu, av_col, rwp):
    hop_spec = pl.BlockSpec((BB * NP, D2), lambda i: (i, 0))
    return pl.pallas_call(
        _tc_gat_body,
        grid=(B // BB,),
        in_specs=[
            hop_spec, hop_spec, hop_spec,
            pl.BlockSpec((BB, D2), lambda i: (i, 0)),
            pl.BlockSpec((D2, D), lambda i: (0, 0)),
            pl.BlockSpec((D2, D), lambda i: (0, 0)),
            pl.BlockSpec((D, 1), lambda i: (0, 0)),
            pl.BlockSpec((4, D2, D), lambda i: (0, 0, 0)),
        ],
        out_specs=pl.BlockSpec((BB, D), lambda i: (i, 0)),
        out_shape=jax.ShapeDtypeStruct((B, D), jnp.float32),
    )(g1, g2, g3, gt, awi, awu, av_col, rwp)


def kernel(target_ids, support_1st, support_2nd, support_3rd,
           user_emb, item_emb, att_w, att_v, refine_w):
    combo = jnp.concatenate([item_emb, user_emb], axis=1)

    def pad_idx(s):
        return jnp.concatenate([s, s[:, :NP - N]], axis=1).reshape(
            NW, N_CHUNKS, CHUNK)

    i1idx = pad_idx(support_1st)
    uidx = pad_idx(support_2nd)
    i3idx = pad_idx(support_3rd)

    g1, g2, g3, gt = _make_sc_gather()(combo, i1idx, uidx, i3idx, target_ids)

    z = jnp.zeros((D, D), jnp.float32)
    awi = jnp.concatenate([att_w, z], axis=0)
    awu = jnp.concatenate([z, att_w], axis=0)
    rwp = jnp.stack([
        jnp.concatenate([refine_w[0:D], z], axis=0),
        jnp.concatenate([z, refine_w[D:2 * D]], axis=0),
        jnp.concatenate([refine_w[2 * D:3 * D], z], axis=0),
        jnp.concatenate([z, refine_w[3 * D:4 * D]], axis=0),
    ])
    return _tc_gat(g1, g2, g3, gt, awi, awu, att_v.reshape(D, 1), rwp)
